# Initial kernel scaffold; baseline (speedup 1.0000x reference)
import jax
import jax.numpy as jnp
from jax.experimental import pallas as pl


def kernel(x, edge_attr, edges, params):
    raise NotImplementedError("write your pallas kernel here")


# speedup vs baseline: 1.7137x; 1.7137x over previous
import functools

import jax
import jax.numpy as jnp
from jax import lax
from jax.experimental import pallas as pl
from jax.experimental.pallas import tpu as pltpu
from jax.experimental.pallas import tpu_sc as plsc

N = 10000
E = 320000
H = 128
OUT = 64
EDIM = 264

NC = 2
NS = 16
NW = NC * NS
EPW = E // NW
CH = 80
NCHUNK = EPW // CH
CHG = 40
NCHUNKG = EPW // CHG
ASTRIDE = 624
AWIN = 640
ZROWS = 128
CNTW = 16

TE = 1280
TN = 1000

_MESH = plsc.VectorSubcoreMesh(core_axis_name="c", subcore_axis_name="s")


def _silu(v):
    return v / (1.0 + jnp.exp(-v))


@functools.partial(
    pl.kernel,
    mesh=_MESH,
    out_type=(
        jax.ShapeDtypeStruct((E, H), jnp.float32),
        jax.ShapeDtypeStruct((E, H), jnp.float32),
    ),
    scratch_types=[
        pltpu.VMEM((CHG,), jnp.int32),
        pltpu.VMEM((CHG,), jnp.int32),
        pltpu.VMEM((CHG, H), jnp.float32),
        pltpu.VMEM((CHG, H), jnp.float32),
        pltpu.SemaphoreType.DMA,
        pltpu.SemaphoreType.DMA,
    ],
)
def _sc_gather(xs_hbm, xr_hbm, send_hbm, recv_hbm, gs_hbm, gr_hbm,
               idx_s, idx_r, buf_s, buf_r, sem_s, sem_r):
    wid = lax.axis_index("s") * NC + lax.axis_index("c")
    base = wid * EPW

    def chunk(k, carry):
        off = base + k * CHG
        pltpu.sync_copy(send_hbm.at[pl.ds(off, CHG)], idx_s)
        pltpu.sync_copy(recv_hbm.at[pl.ds(off, CHG)], idx_r)
        cs = pltpu.async_copy(xs_hbm.at[idx_s], buf_s, sem_s)
        cr = pltpu.async_copy(xr_hbm.at[idx_r], buf_r, sem_r)
        cs.wait()
        cr.wait()
        pltpu.sync_copy(buf_s, gs_hbm.at[pl.ds(off, CHG)])
        pltpu.sync_copy(buf_r, gr_hbm.at[pl.ds(off, CHG)])
        return carry

    lax.fori_loop(0, NCHUNKG, chunk, 0)


def _fill_const(ref, rows, val):
    def fill(i, carry):
        for j in range(H // 16):
            ref[i, pl.ds(16 * j, 16)] = jnp.full((16,), val, jnp.float32)
        return carry
    lax.fori_loop(0, rows, fill, 0)


@functools.partial(
    pl.kernel,
    mesh=_MESH,
    out_type=jax.ShapeDtypeStruct((NC * N, H), jnp.float32),
    scratch_types=[
        pltpu.VMEM((CH,), jnp.int32),
        pltpu.VMEM((CH, H), jnp.float32),
        pltpu.VMEM((ZROWS, H), jnp.float32),
        pltpu.VMEM_SHARED((N, H), jnp.float32),
    ],
)
def _sc_scatter(h_hbm, recv_hbm, sums_hbm, idx_v, hbuf, zbuf, acc):
    cid = lax.axis_index("c")
    sid = lax.axis_index("s")
    base = (sid * NC + cid) * EPW

    _fill_const(zbuf, ZROWS, 0.0)
    win0 = sid * ASTRIDE
    def zero_slice(t, carry):
        pltpu.sync_copy(zbuf, acc.at[pl.ds(win0 + t * ZROWS, ZROWS)])
        return carry
    lax.fori_loop(0, AWIN // ZROWS, zero_slice, 0)
    plsc.subcore_barrier()

    def chunk(k, carry):
        off = base + k * CH
        pltpu.sync_copy(recv_hbm.at[pl.ds(off, CH)], idx_v)
        pltpu.sync_copy(h_hbm.at[pl.ds(off, CH)], hbuf)
        pltpu.sync_copy(hbuf, acc.at[idx_v], add=True)
        return carry
    lax.fori_loop(0, NCHUNK, chunk, 0)
    plsc.subcore_barrier()

    pltpu.sync_copy(acc.at[pl.ds(win0, AWIN)],
                    sums_hbm.at[pl.ds(cid * N + win0, AWIN)])


@functools.partial(
    pl.kernel,
    mesh=_MESH,
    out_type=jax.ShapeDtypeStruct((NC * N, H), jnp.float32),
    scratch_types=[
        pltpu.VMEM((CH,), jnp.int32),
        pltpu.VMEM((CH, H), jnp.float32),
        pltpu.VMEM((ZROWS, H), jnp.float32),
        pltpu.VMEM_SHARED((N, H), jnp.float32),
    ],
)
def _sc_counts(recv_hbm, cnt_hbm, idx_v, onesb, zbuf, acc):
    cid = lax.axis_index("c")
    sid = lax.axis_index("s")
    base = (sid * NC + cid) * EPW

    _fill_const(zbuf, ZROWS, 0.0)
    _fill_const(onesb, CH, 1.0)
    win0 = sid * ASTRIDE
    def zero_slice(t, carry):
        pltpu.sync_copy(zbuf, acc.at[pl.ds(win0 + t * ZROWS, ZROWS)])
        return carry
    lax.fori_loop(0, AWIN // ZROWS, zero_slice, 0)
    plsc.subcore_barrier()

    def chunk(k, carry):
        off = base + k * CH
        pltpu.sync_copy(recv_hbm.at[pl.ds(off, CH)], idx_v)
        pltpu.sync_copy(onesb, acc.at[idx_v], add=True)
        return carry
    lax.fori_loop(0, NCHUNK, chunk, 0)
    plsc.subcore_barrier()

    pltpu.sync_copy(acc.at[pl.ds(win0, AWIN)],
                    cnt_hbm.at[pl.ds(cid * N + win0, AWIN)])


def _edge1_body(ea_ref, w1_ref, b1_ref, w2_ref, b2_ref, o_ref):
    t = jnp.dot(ea_ref[...], w1_ref[...],
                preferred_element_type=jnp.float32) + b1_ref[...]
    t = _silu(t)
    o_ref[...] = _silu(jnp.dot(t, w2_ref[...],
                               preferred_element_type=jnp.float32) + b2_ref[...])


def _edge_mlp1(ea, w1, b1, w2, b2):
    return pl.pallas_call(
        _edge1_body,
        grid=(E // TE,),
        in_specs=[
            pl.BlockSpec((TE, EDIM), lambda i: (i, 0)),
            pl.BlockSpec((EDIM, H), lambda i: (0, 0)),
            pl.BlockSpec((1, H), lambda i: (0, 0)),
            pl.BlockSpec((H, H), lambda i: (0, 0)),
            pl.BlockSpec((1, H), lambda i: (0, 0)),
        ],
        out_specs=pl.BlockSpec((TE, H), lambda i: (i, 0)),
        out_shape=jax.ShapeDtypeStruct((E, H), jnp.float32),
    )(ea, w1, b1, w2, b2)


def _edgeN_body(gs_ref, gr_ref, hp_ref, we_ref, b1_ref, w2_ref, b2_ref, o_ref):
    t = gs_ref[...] + gr_ref[...] + jnp.dot(
        hp_ref[...], we_ref[...], preferred_element_type=jnp.float32) + b1_ref[...]
    t = _silu(t)
    o_ref[...] = _silu(jnp.dot(t, w2_ref[...],
                               preferred_element_type=jnp.float32) + b2_ref[...])


def _edge_mlpN(gs, gr, hp, we, b1, w2, b2):
    return pl.pallas_call(
        _edgeN_body,
        grid=(E // TE,),
        in_specs=[
            pl.BlockSpec((TE, H), lambda i: (i, 0)),
            pl.BlockSpec((TE, H), lambda i: (i, 0)),
            pl.BlockSpec((TE, H), lambda i: (i, 0)),
            pl.BlockSpec((H, H), lambda i: (0, 0)),
            pl.BlockSpec((1, H), lambda i: (0, 0)),
            pl.BlockSpec((H, H), lambda i: (0, 0)),
            pl.BlockSpec((1, H), lambda i: (0, 0)),
        ],
        out_specs=pl.BlockSpec((TE, H), lambda i: (i, 0)),
        out_shape=jax.ShapeDtypeStruct((E, H), jnp.float32),
    )(gs, gr, hp, we, b1, w2, b2)


def _node_mid_body(x_ref, s0_ref, s1_ref, c_ref, u1w_ref, u1b_ref,
                   u2w_ref, u2b_ref, ws_ref, wr_ref,
                   oxn_ref, oxs_ref, oxr_ref):
    c = jnp.maximum(c_ref[...][:, 0:1], 1.0)
    xm = x_ref[...] + (s0_ref[...] + s1_ref[...]) / c
    u = _silu(jnp.dot(xm, u1w_ref[...],
                      preferred_element_type=jnp.float32) + u1b_ref[...])
    xn = xm + jnp.dot(u, u2w_ref[...],
                      preferred_element_type=jnp.float32) + u2b_ref[...]
    oxn_ref[...] = xn
    oxs_ref[...] = jnp.dot(xn, ws_ref[...], preferred_element_type=jnp.float32)
    oxr_ref[...] = jnp.dot(xn, wr_ref[...], preferred_element_type=jnp.float32)


def _node_mid(x, s0, s1, cnt, u1w, u1b, u2w, u2b, ws, wr):
    return pl.pallas_call(
        _node_mid_body,
        grid=(N // TN,),
        in_specs=[
            pl.BlockSpec((TN, H), lambda i: (i, 0)),
            pl.BlockSpec((TN, H), lambda i: (i, 0)),
            pl.BlockSpec((TN, H), lambda i: (i, 0)),
            pl.BlockSpec((TN, H), lambda i: (i, 0)),
            pl.BlockSpec((H, 2 * H), lambda i: (0, 0)),
            pl.BlockSpec((1, 2 * H), lambda i: (0, 0)),
            pl.BlockSpec((2 * H, H), lambda i: (0, 0)),
            pl.BlockSpec((1, H), lambda i: (0, 0)),
            pl.BlockSpec((H, H), lambda i: (0, 0)),
            pl.BlockSpec((H, H), lambda i: (0, 0)),
        ],
        out_specs=[
            pl.BlockSpec((TN, H), lambda i: (i, 0)),
            pl.BlockSpec((TN, H), lambda i: (i, 0)),
            pl.BlockSpec((TN, H), lambda i: (i, 0)),
        ],
        out_shape=[
            jax.ShapeDtypeStruct((N, H), jnp.float32),
            jax.ShapeDtypeStruct((N, H), jnp.float32),
            jax.ShapeDtypeStruct((N, H), jnp.float32),
        ],
    )(x, s0, s1, cnt, u1w, u1b, u2w, u2b, ws, wr)


def _node_last_body(x_ref, s0_ref, s1_ref, c_ref, u1w_ref, u1b_ref,
                    u2w_ref, u2b_ref, w1_ref, b1_ref, w2_ref, b2_ref,
                    w3_ref, b3_ref, o_ref):
    c = jnp.maximum(c_ref[...][:, 0:1], 1.0)
    xm = x_ref[...] + (s0_ref[...] + s1_ref[...]) / c
    u = _silu(jnp.dot(xm, u1w_ref[...],
                      preferred_element_type=jnp.float32) + u1b_ref[...])
    xn = xm + jnp.dot(u, u2w_ref[...],
                      preferred_element_type=jnp.float32) + u2b_ref[...]
    t = _silu(jnp.dot(xn, w1_ref[...],
                      preferred_element_type=jnp.float32) + b1_ref[...])
    t = _silu(jnp.dot(t, w2_ref[...],
                      preferred_element_type=jnp.float32) + b2_ref[...])
    o_ref[...] = jnp.dot(t, w3_ref[...],
                         preferred_element_type=jnp.float32) + b3_ref[...]


def _node_last(x, s0, s1, cnt, u1w, u1b, u2w, u2b, w1, b1, w2, b2, w3, b3):
    return pl.pallas_call(
        _node_last_body,
        grid=(N // TN,),
        in_specs=[
            pl.BlockSpec((TN, H), lambda i: (i, 0)),
            pl.BlockSpec((TN, H), lambda i: (i, 0)),
            pl.BlockSpec((TN, H), lambda i: (i, 0)),
            pl.BlockSpec((TN, H), lambda i: (i, 0)),
            pl.BlockSpec((H, 2 * H), lambda i: (0, 0)),
            pl.BlockSpec((1, 2 * H), lambda i: (0, 0)),
            pl.BlockSpec((2 * H, H), lambda i: (0, 0)),
            pl.BlockSpec((1, H), lambda i: (0, 0)),
            pl.BlockSpec((H, H), lambda i: (0, 0)),
            pl.BlockSpec((1, H), lambda i: (0, 0)),
            pl.BlockSpec((H, H), lambda i: (0, 0)),
            pl.BlockSpec((1, H), lambda i: (0, 0)),
            pl.BlockSpec((H, OUT), lambda i: (0, 0)),
            pl.BlockSpec((1, OUT), lambda i: (0, 0)),
        ],
        out_specs=pl.BlockSpec((TN, OUT), lambda i: (i, 0)),
        out_shape=jax.ShapeDtypeStruct((N, OUT), jnp.float32),
    )(x, s0, s1, cnt, u1w, u1b, u2w, u2b, w1, b1, w2, b2, w3, b3)


def kernel(x, edge_attr, edges, params):
    send, recv = edges[0], edges[1]
    p1 = params["l1"]
    plist = [params["l2"], params["l3"], params["l4"]]
    po = params["out"]

    def r1(b):
        return b.reshape(1, -1)

    h = _edge_mlp1(edge_attr, p1["m1w"], r1(p1["m1b"]),
                   p1["m2w"], r1(p1["m2b"]))
    cnts = _sc_counts(recv)
    cnt = cnts[:N] + cnts[N:]
    sums = _sc_scatter(h, recv)
    s0, s1 = sums[:N], sums[N:]

    x_cur = x
    ulayer = p1
    for pn in plist:
        ws, wr, we = (pn["m1w"][:H], pn["m1w"][H:2 * H], pn["m1w"][2 * H:])
        x_cur, xs, xr = _node_mid(x_cur, s0, s1, cnt,
                                  ulayer["u1w"], r1(ulayer["u1b"]),
                                  ulayer["u2w"], r1(ulayer["u2b"]), ws, wr)
        gs, gr = _sc_gather(xs, xr, send, recv)
        h = _edge_mlpN(gs, gr, h, we, r1(pn["m1b"]),
                       pn["m2w"], r1(pn["m2b"]))
        sums = _sc_scatter(h, recv)
        s0, s1 = sums[:N], sums[N:]
        ulayer = pn

    return _node_last(x_cur, s0, s1, cnt,
                      ulayer["u1w"], r1(ulayer["u1b"]),
                      ulayer["u2w"], r1(ulayer["u2b"]),
                      po["w1"], r1(po["b1"]), po["w2"], r1(po["b2"]),
                      po["w3"], r1(po["b3"]))


# speedup vs baseline: 2.6875x; 1.5682x over previous
import functools

import jax
import jax.numpy as jnp
from jax import lax
from jax.experimental import pallas as pl
from jax.experimental.pallas import tpu as pltpu
from jax.experimental.pallas import tpu_sc as plsc

N = 10000
E = 320000
H = 128
OUT = 64
EDIM = 264

NC = 2
NS = 16
NW = NC * NS
EPW = E // NW
CH = 80
NCHUNK = EPW // CH
RING = 5
NGRP = NCHUNK // RING
SCH = 40
SNCHUNK = EPW // SCH
SGRP = SNCHUNK // RING
ASTRIDE = 624
AWIN = 640
ZROWS = 64
CNTW = 16

TE = 1280
TN = 1000

_MESH = plsc.VectorSubcoreMesh(core_axis_name="c", subcore_axis_name="s")


def _silu(v):
    return v / (1.0 + jnp.exp(-v))


_NOTILE = pltpu.CompilerParams(use_tc_tiling_on_sc=False)


@functools.partial(
    pl.kernel,
    mesh=_MESH,
    out_type=(
        jax.ShapeDtypeStruct((E, H), jnp.float32),
        jax.ShapeDtypeStruct((E, H), jnp.float32),
    ),
    scratch_types=(
        [pltpu.VMEM((EPW,), jnp.int32)] * 2
        + [pltpu.VMEM((CH, H), jnp.float32)] * (2 * RING)
        + [pltpu.SemaphoreType.DMA] * (4 * RING)
    ),
    compiler_params=_NOTILE,
)
def _sc_gather(xs_hbm, xr_hbm, send_hbm, recv_hbm, gs_hbm, gr_hbm, *scr):
    idx_s, idx_r = scr[0], scr[1]
    buf_s = scr[2:2 + RING]
    buf_r = scr[2 + RING:2 + 2 * RING]
    sem_g = scr[2 + 2 * RING:2 + 4 * RING]
    sem_c = scr[2 + 4 * RING:2 + 6 * RING]
    wid = lax.axis_index("s") * NC + lax.axis_index("c")
    base = wid * EPW

    pltpu.sync_copy(send_hbm.at[pl.ds(base, EPW)], idx_s)
    pltpu.sync_copy(recv_hbm.at[pl.ds(base, EPW)], idx_r)

    def _gather(k, b):
        pltpu.async_copy(xs_hbm.at[idx_s.at[pl.ds(k * CH, CH)]],
                         buf_s[b], sem_g[b])
        pltpu.async_copy(xr_hbm.at[idx_r.at[pl.ds(k * CH, CH)]],
                         buf_r[b], sem_g[RING + b])

    def _wait(sem, ref):
        pltpu.make_async_copy(gs_hbm.at[pl.ds(0, CH)], ref, sem).wait()

    def group(g, carry):
        for b in range(RING):
            @pl.when(g > 0)
            def _():
                _wait(sem_c[b], buf_s[b])
                _wait(sem_c[RING + b], buf_r[b])
            _gather(g * RING + b, b)
        for b in range(RING):
            k = g * RING + b
            off = base + k * CH
            _wait(sem_g[b], buf_s[b])
            _wait(sem_g[RING + b], buf_r[b])
            pltpu.async_copy(buf_s[b], gs_hbm.at[pl.ds(off, CH)], sem_c[b])
            pltpu.async_copy(buf_r[b], gr_hbm.at[pl.ds(off, CH)],
                             sem_c[RING + b])
        return carry

    lax.fori_loop(0, NGRP, group, 0)
    for b in range(RING):
        _wait(sem_c[b], buf_s[b])
        _wait(sem_c[RING + b], buf_r[b])


def _fill_const(ref, rows, val):
    def fill(i, carry):
        for j in range(H // 16):
            ref[i, pl.ds(16 * j, 16)] = jnp.full((16,), val, jnp.float32)
        return carry
    lax.fori_loop(0, rows, fill, 0)


@functools.partial(
    pl.kernel,
    mesh=_MESH,
    out_type=jax.ShapeDtypeStruct((NC * N, H), jnp.float32),
    scratch_types=(
        [pltpu.VMEM((SCH,), jnp.int32)] * RING
        + [pltpu.VMEM((SCH, H), jnp.float32)] * RING
        + [pltpu.VMEM((ZROWS, H), jnp.float32),
           pltpu.VMEM_SHARED((N, H), jnp.float32)]
        + [pltpu.SemaphoreType.DMA] * (2 * RING)
    ),
    compiler_params=_NOTILE,
)
def _sc_scatter(h_hbm, recv_hbm, sums_hbm, *scr):
    idx_v = scr[0:RING]
    hbuf = scr[RING:2 * RING]
    zbuf = scr[2 * RING]
    acc = scr[1 + 2 * RING]
    sem_l = scr[2 + 2 * RING:2 + 3 * RING]
    sem_w = scr[2 + 3 * RING:2 + 4 * RING]
    cid = lax.axis_index("c")
    sid = lax.axis_index("s")
    base = (sid * NC + cid) * EPW

    _fill_const(zbuf, ZROWS, 0.0)
    win0 = sid * ASTRIDE
    def zero_slice(t, carry):
        pltpu.sync_copy(zbuf, acc.at[pl.ds(win0 + t * ZROWS, ZROWS)])
        return carry
    lax.fori_loop(0, AWIN // ZROWS, zero_slice, 0)
    plsc.subcore_barrier()

    def _wait(sem, ref):
        src = (h_hbm.at[pl.ds(0, SCH)] if len(ref.shape) == 2
               else recv_hbm.at[pl.ds(0, SCH)])
        pltpu.make_async_copy(src, ref, sem).wait()

    def group(g, carry):
        for b in range(RING):
            k = g * RING + b
            @pl.when(g > 0)
            def _():
                _wait(sem_w[b], hbuf[b])
            pltpu.async_copy(recv_hbm.at[pl.ds(base + k * SCH, SCH)],
                             idx_v[b], sem_l[b])
            pltpu.async_copy(h_hbm.at[pl.ds(base + k * SCH, SCH)],
                             hbuf[b], sem_l[b])
        for b in range(RING):
            _wait(sem_l[b], idx_v[b])
            _wait(sem_l[b], hbuf[b])
            pltpu.async_copy(hbuf[b], acc.at[idx_v[b]], sem_w[b], add=True)
        return carry
    lax.fori_loop(0, SGRP, group, 0)
    for b in range(RING):
        _wait(sem_w[b], hbuf[b])
    plsc.subcore_barrier()

    pltpu.sync_copy(acc.at[pl.ds(win0, AWIN)],
                    sums_hbm.at[pl.ds(cid * N + win0, AWIN)])


@functools.partial(
    pl.kernel,
    mesh=_MESH,
    out_type=jax.ShapeDtypeStruct((NC * N, H), jnp.float32),
    scratch_types=[
        pltpu.VMEM((CH,), jnp.int32),
        pltpu.VMEM((CH, H), jnp.float32),
        pltpu.VMEM((ZROWS, H), jnp.float32),
        pltpu.VMEM_SHARED((N, H), jnp.float32),
    ],
)
def _sc_counts(recv_hbm, cnt_hbm, idx_v, onesb, zbuf, acc):
    cid = lax.axis_index("c")
    sid = lax.axis_index("s")
    base = (sid * NC + cid) * EPW

    _fill_const(zbuf, ZROWS, 0.0)
    _fill_const(onesb, CH, 1.0)
    win0 = sid * ASTRIDE
    def zero_slice(t, carry):
        pltpu.sync_copy(zbuf, acc.at[pl.ds(win0 + t * ZROWS, ZROWS)])
        return carry
    lax.fori_loop(0, AWIN // ZROWS, zero_slice, 0)
    plsc.subcore_barrier()

    def chunk(k, carry):
        off = base + k * CH
        pltpu.sync_copy(recv_hbm.at[pl.ds(off, CH)], idx_v)
        pltpu.sync_copy(onesb, acc.at[idx_v], add=True)
        return carry
    lax.fori_loop(0, NCHUNK, chunk, 0)
    plsc.subcore_barrier()

    pltpu.sync_copy(acc.at[pl.ds(win0, AWIN)],
                    cnt_hbm.at[pl.ds(cid * N + win0, AWIN)])


def _edge1_body(ea_ref, w1_ref, b1_ref, w2_ref, b2_ref, o_ref):
    t = jnp.dot(ea_ref[...], w1_ref[...],
                preferred_element_type=jnp.float32) + b1_ref[...]
    t = _silu(t)
    o_ref[...] = _silu(jnp.dot(t, w2_ref[...],
                               preferred_element_type=jnp.float32) + b2_ref[...])


def _edge_mlp1(ea, w1, b1, w2, b2):
    return pl.pallas_call(
        _edge1_body,
        grid=(E // TE,),
        in_specs=[
            pl.BlockSpec((TE, EDIM), lambda i: (i, 0)),
            pl.BlockSpec((EDIM, H), lambda i: (0, 0)),
            pl.BlockSpec((1, H), lambda i: (0, 0)),
            pl.BlockSpec((H, H), lambda i: (0, 0)),
            pl.BlockSpec((1, H), lambda i: (0, 0)),
        ],
        out_specs=pl.BlockSpec((TE, H), lambda i: (i, 0)),
        out_shape=jax.ShapeDtypeStruct((E, H), jnp.float32),
    )(ea, w1, b1, w2, b2)


def _edgeN_body(gs_ref, gr_ref, hp_ref, we_ref, b1_ref, w2_ref, b2_ref, o_ref):
    t = gs_ref[...] + gr_ref[...] + jnp.dot(
        hp_ref[...], we_ref[...], preferred_element_type=jnp.float32) + b1_ref[...]
    t = _silu(t)
    o_ref[...] = _silu(jnp.dot(t, w2_ref[...],
                               preferred_element_type=jnp.float32) + b2_ref[...])


def _edge_mlpN(gs, gr, hp, we, b1, w2, b2):
    return pl.pallas_call(
        _edgeN_body,
        grid=(E // TE,),
        in_specs=[
            pl.BlockSpec((TE, H), lambda i: (i, 0)),
            pl.BlockSpec((TE, H), lambda i: (i, 0)),
            pl.BlockSpec((TE, H), lambda i: (i, 0)),
            pl.BlockSpec((H, H), lambda i: (0, 0)),
            pl.BlockSpec((1, H), lambda i: (0, 0)),
            pl.BlockSpec((H, H), lambda i: (0, 0)),
            pl.BlockSpec((1, H), lambda i: (0, 0)),
        ],
        out_specs=pl.BlockSpec((TE, H), lambda i: (i, 0)),
        out_shape=jax.ShapeDtypeStruct((E, H), jnp.float32),
    )(gs, gr, hp, we, b1, w2, b2)


def _node_mid_body(x_ref, s0_ref, s1_ref, c_ref, u1w_ref, u1b_ref,
                   u2w_ref, u2b_ref, ws_ref, wr_ref,
                   oxn_ref, oxs_ref, oxr_ref):
    c = jnp.maximum(c_ref[...][:, 0:1], 1.0)
    xm = x_ref[...] + (s0_ref[...] + s1_ref[...]) / c
    u = _silu(jnp.dot(xm, u1w_ref[...],
                      preferred_element_type=jnp.float32) + u1b_ref[...])
    xn = xm + jnp.dot(u, u2w_ref[...],
                      preferred_element_type=jnp.float32) + u2b_ref[...]
    oxn_ref[...] = xn
    oxs_ref[...] = jnp.dot(xn, ws_ref[...], preferred_element_type=jnp.float32)
    oxr_ref[...] = jnp.dot(xn, wr_ref[...], preferred_element_type=jnp.float32)


def _node_mid(x, s0, s1, cnt, u1w, u1b, u2w, u2b, ws, wr):
    return pl.pallas_call(
        _node_mid_body,
        grid=(N // TN,),
        in_specs=[
            pl.BlockSpec((TN, H), lambda i: (i, 0)),
            pl.BlockSpec((TN, H), lambda i: (i, 0)),
            pl.BlockSpec((TN, H), lambda i: (i, 0)),
            pl.BlockSpec((TN, H), lambda i: (i, 0)),
            pl.BlockSpec((H, 2 * H), lambda i: (0, 0)),
            pl.BlockSpec((1, 2 * H), lambda i: (0, 0)),
            pl.BlockSpec((2 * H, H), lambda i: (0, 0)),
            pl.BlockSpec((1, H), lambda i: (0, 0)),
            pl.BlockSpec((H, H), lambda i: (0, 0)),
            pl.BlockSpec((H, H), lambda i: (0, 0)),
        ],
        out_specs=[
            pl.BlockSpec((TN, H), lambda i: (i, 0)),
            pl.BlockSpec((TN, H), lambda i: (i, 0)),
            pl.BlockSpec((TN, H), lambda i: (i, 0)),
        ],
        out_shape=[
            jax.ShapeDtypeStruct((N, H), jnp.float32),
            jax.ShapeDtypeStruct((N, H), jnp.float32),
            jax.ShapeDtypeStruct((N, H), jnp.float32),
        ],
    )(x, s0, s1, cnt, u1w, u1b, u2w, u2b, ws, wr)


def _node_last_body(x_ref, s0_ref, s1_ref, c_ref, u1w_ref, u1b_ref,
                    u2w_ref, u2b_ref, w1_ref, b1_ref, w2_ref, b2_ref,
                    w3_ref, b3_ref, o_ref):
    c = jnp.maximum(c_ref[...][:, 0:1], 1.0)
    xm = x_ref[...] + (s0_ref[...] + s1_ref[...]) / c
    u = _silu(jnp.dot(xm, u1w_ref[...],
                      preferred_element_type=jnp.float32) + u1b_ref[...])
    xn = xm + jnp.dot(u, u2w_ref[...],
                      preferred_element_type=jnp.float32) + u2b_ref[...]
    t = _silu(jnp.dot(xn, w1_ref[...],
                      preferred_element_type=jnp.float32) + b1_ref[...])
    t = _silu(jnp.dot(t, w2_ref[...],
                      preferred_element_type=jnp.float32) + b2_ref[...])
    o_ref[...] = jnp.dot(t, w3_ref[...],
                         preferred_element_type=jnp.float32) + b3_ref[...]


def _node_last(x, s0, s1, cnt, u1w, u1b, u2w, u2b, w1, b1, w2, b2, w3, b3):
    return pl.pallas_call(
        _node_last_body,
        grid=(N // TN,),
        in_specs=[
            pl.BlockSpec((TN, H), lambda i: (i, 0)),
            pl.BlockSpec((TN, H), lambda i: (i, 0)),
            pl.BlockSpec((TN, H), lambda i: (i, 0)),
            pl.BlockSpec((TN, H), lambda i: (i, 0)),
            pl.BlockSpec((H, 2 * H), lambda i: (0, 0)),
            pl.BlockSpec((1, 2 * H), lambda i: (0, 0)),
            pl.BlockSpec((2 * H, H), lambda i: (0, 0)),
            pl.BlockSpec((1, H), lambda i: (0, 0)),
            pl.BlockSpec((H, H), lambda i: (0, 0)),
            pl.BlockSpec((1, H), lambda i: (0, 0)),
            pl.BlockSpec((H, H), lambda i: (0, 0)),
            pl.BlockSpec((1, H), lambda i: (0, 0)),
            pl.BlockSpec((H, OUT), lambda i: (0, 0)),
            pl.BlockSpec((1, OUT), lambda i: (0, 0)),
        ],
        out_specs=pl.BlockSpec((TN, OUT), lambda i: (i, 0)),
        out_shape=jax.ShapeDtypeStruct((N, OUT), jnp.float32),
    )(x, s0, s1, cnt, u1w, u1b, u2w, u2b, w1, b1, w2, b2, w3, b3)


def kernel(x, edge_attr, edges, params):
    send, recv = edges[0], edges[1]
    p1 = params["l1"]
    plist = [params["l2"], params["l3"], params["l4"]]
    po = params["out"]

    def r1(b):
        return b.reshape(1, -1)

    h = _edge_mlp1(edge_attr, p1["m1w"], r1(p1["m1b"]),
                   p1["m2w"], r1(p1["m2b"]))
    cnts = _sc_counts(recv)
    cnt = cnts[:N] + cnts[N:]
    sums = _sc_scatter(h, recv)
    s0, s1 = sums[:N], sums[N:]

    x_cur = x
    ulayer = p1
    for pn in plist:
        ws, wr, we = (pn["m1w"][:H], pn["m1w"][H:2 * H], pn["m1w"][2 * H:])
        x_cur, xs, xr = _node_mid(x_cur, s0, s1, cnt,
                                  ulayer["u1w"], r1(ulayer["u1b"]),
                                  ulayer["u2w"], r1(ulayer["u2b"]), ws, wr)
        gs, gr = _sc_gather(xs, xr, send, recv)
        h = _edge_mlpN(gs, gr, h, we, r1(pn["m1b"]),
                       pn["m2w"], r1(pn["m2b"]))
        sums = _sc_scatter(h, recv)
        s0, s1 = sums[:N], sums[N:]
        ulayer = pn

    return _node_last(x_cur, s0, s1, cnt,
                      ulayer["u1w"], r1(ulayer["u1b"]),
                      ulayer["u2w"], r1(ulayer["u2b"]),
                      po["w1"], r1(po["b1"]), po["w2"], r1(po["b2"]),
                      po["w3"], r1(po["b3"]))


# speedup vs baseline: 2.9014x; 1.0796x over previous
import functools

import jax
import jax.numpy as jnp
from jax import lax
from jax.experimental import pallas as pl
from jax.experimental.pallas import tpu as pltpu
from jax.experimental.pallas import tpu_sc as plsc

N = 10000
E = 320000
H = 128
OUT = 64
EDIM = 264

NC = 2
NS = 16
NW = NC * NS
EPW = E // NW
CH = 80
NCHUNK = EPW // CH
RING = 5
NGRP = NCHUNK // RING
SCH = 40
SNCHUNK = EPW // SCH
SGRP = SNCHUNK // RING
ASTRIDE = 624
AWIN = 640
ZROWS = 64
CNTW = 16

TE = 1280
TN = 1000

_MESH = plsc.VectorSubcoreMesh(core_axis_name="c", subcore_axis_name="s")


def _silu(v):
    return v / (1.0 + jnp.exp(-v))


_NOTILE = pltpu.CompilerParams(use_tc_tiling_on_sc=False)


@functools.partial(
    pl.kernel,
    mesh=_MESH,
    out_type=jax.ShapeDtypeStruct((E, H), jnp.float32),
    scratch_types=(
        [pltpu.VMEM((EPW,), jnp.int32)] * 2
        + [pltpu.VMEM((CH, H), jnp.float32)] * (2 * RING)
        + [pltpu.SemaphoreType.DMA] * (3 * RING)
    ),
    compiler_params=_NOTILE,
)
def _sc_gather(xs_hbm, xr_hbm, send_hbm, recv_hbm, g_hbm, *scr):
    idx_s, idx_r = scr[0], scr[1]
    buf_s = scr[2:2 + RING]
    buf_r = scr[2 + RING:2 + 2 * RING]
    sem_g = scr[2 + 2 * RING:2 + 4 * RING]
    sem_c = scr[2 + 4 * RING:2 + 5 * RING]
    wid = lax.axis_index("s") * NC + lax.axis_index("c")
    base = wid * EPW

    pltpu.sync_copy(send_hbm.at[pl.ds(base, EPW)], idx_s)
    pltpu.sync_copy(recv_hbm.at[pl.ds(base, EPW)], idx_r)

    def _gather(k, b):
        pltpu.async_copy(xs_hbm.at[idx_s.at[pl.ds(k * CH, CH)]],
                         buf_s[b], sem_g[b])
        pltpu.async_copy(xr_hbm.at[idx_r.at[pl.ds(k * CH, CH)]],
                         buf_r[b], sem_g[RING + b])

    def _wait(sem, ref):
        pltpu.make_async_copy(g_hbm.at[pl.ds(0, CH)], ref, sem).wait()

    def group(g, carry):
        for b in range(RING):
            @pl.when(g > 0)
            def _():
                _wait(sem_c[b], buf_s[b])
            _gather(g * RING + b, b)
        for b in range(RING):
            k = g * RING + b
            off = base + k * CH
            _wait(sem_g[b], buf_s[b])
            _wait(sem_g[RING + b], buf_r[b])
            def add_row(r, carry2):
                for j in range(H // 16):
                    sl = pl.ds(16 * j, 16)
                    buf_s[b][r, sl] = buf_s[b][r, sl] + buf_r[b][r, sl]
                return carry2
            lax.fori_loop(0, CH, add_row, 0)
            pltpu.async_copy(buf_s[b], g_hbm.at[pl.ds(off, CH)], sem_c[b])
        return carry

    lax.fori_loop(0, NGRP, group, 0)
    for b in range(RING):
        _wait(sem_c[b], buf_s[b])


def _fill_const(ref, rows, val):
    def fill(i, carry):
        for j in range(H // 16):
            ref[i, pl.ds(16 * j, 16)] = jnp.full((16,), val, jnp.float32)
        return carry
    lax.fori_loop(0, rows, fill, 0)


@functools.partial(
    pl.kernel,
    mesh=_MESH,
    out_type=jax.ShapeDtypeStruct((NC * N, H), jnp.float32),
    scratch_types=(
        [pltpu.VMEM((SCH,), jnp.int32)] * RING
        + [pltpu.VMEM((SCH, H), jnp.float32)] * RING
        + [pltpu.VMEM((ZROWS, H), jnp.float32),
           pltpu.VMEM_SHARED((N, H), jnp.float32)]
        + [pltpu.SemaphoreType.DMA] * (2 * RING)
    ),
    compiler_params=_NOTILE,
)
def _sc_scatter(h_hbm, recv_hbm, sums_hbm, *scr):
    idx_v = scr[0:RING]
    hbuf = scr[RING:2 * RING]
    zbuf = scr[2 * RING]
    acc = scr[1 + 2 * RING]
    sem_l = scr[2 + 2 * RING:2 + 3 * RING]
    sem_w = scr[2 + 3 * RING:2 + 4 * RING]
    cid = lax.axis_index("c")
    sid = lax.axis_index("s")
    base = (sid * NC + cid) * EPW

    _fill_const(zbuf, ZROWS, 0.0)
    win0 = sid * ASTRIDE
    def zero_slice(t, carry):
        pltpu.sync_copy(zbuf, acc.at[pl.ds(win0 + t * ZROWS, ZROWS)])
        return carry
    lax.fori_loop(0, AWIN // ZROWS, zero_slice, 0)
    plsc.subcore_barrier()

    def _wait(sem, ref):
        src = (h_hbm.at[pl.ds(0, SCH)] if len(ref.shape) == 2
               else recv_hbm.at[pl.ds(0, SCH)])
        pltpu.make_async_copy(src, ref, sem).wait()

    def group(g, carry):
        for b in range(RING):
            k = g * RING + b
            @pl.when(g > 0)
            def _():
                _wait(sem_w[b], hbuf[b])
            pltpu.async_copy(recv_hbm.at[pl.ds(base + k * SCH, SCH)],
                             idx_v[b], sem_l[b])
            pltpu.async_copy(h_hbm.at[pl.ds(base + k * SCH, SCH)],
                             hbuf[b], sem_l[b])
        for b in range(RING):
            _wait(sem_l[b], idx_v[b])
            _wait(sem_l[b], hbuf[b])
            pltpu.async_copy(hbuf[b], acc.at[idx_v[b]], sem_w[b], add=True)
        return carry
    lax.fori_loop(0, SGRP, group, 0)
    for b in range(RING):
        _wait(sem_w[b], hbuf[b])
    plsc.subcore_barrier()

    pltpu.sync_copy(acc.at[pl.ds(win0, AWIN)],
                    sums_hbm.at[pl.ds(cid * N + win0, AWIN)])


@functools.partial(
    pl.kernel,
    mesh=_MESH,
    out_type=jax.ShapeDtypeStruct((NC * N, H), jnp.float32),
    scratch_types=[
        pltpu.VMEM((CH,), jnp.int32),
        pltpu.VMEM((CH, H), jnp.float32),
        pltpu.VMEM((ZROWS, H), jnp.float32),
        pltpu.VMEM_SHARED((N, H), jnp.float32),
    ],
)
def _sc_counts(recv_hbm, cnt_hbm, idx_v, onesb, zbuf, acc):
    cid = lax.axis_index("c")
    sid = lax.axis_index("s")
    base = (sid * NC + cid) * EPW

    _fill_const(zbuf, ZROWS, 0.0)
    _fill_const(onesb, CH, 1.0)
    win0 = sid * ASTRIDE
    def zero_slice(t, carry):
        pltpu.sync_copy(zbuf, acc.at[pl.ds(win0 + t * ZROWS, ZROWS)])
        return carry
    lax.fori_loop(0, AWIN // ZROWS, zero_slice, 0)
    plsc.subcore_barrier()

    def chunk(k, carry):
        off = base + k * CH
        pltpu.sync_copy(recv_hbm.at[pl.ds(off, CH)], idx_v)
        pltpu.sync_copy(onesb, acc.at[idx_v], add=True)
        return carry
    lax.fori_loop(0, NCHUNK, chunk, 0)
    plsc.subcore_barrier()

    pltpu.sync_copy(acc.at[pl.ds(win0, AWIN)],
                    cnt_hbm.at[pl.ds(cid * N + win0, AWIN)])


def _edge1_body(ea_ref, w1_ref, b1_ref, w2_ref, b2_ref, o_ref):
    t = jnp.dot(ea_ref[...], w1_ref[...],
                preferred_element_type=jnp.float32) + b1_ref[...]
    t = _silu(t)
    o_ref[...] = _silu(jnp.dot(t, w2_ref[...],
                               preferred_element_type=jnp.float32) + b2_ref[...])


def _edge_mlp1(ea, w1, b1, w2, b2):
    return pl.pallas_call(
        _edge1_body,
        grid=(E // TE,),
        in_specs=[
            pl.BlockSpec((TE, EDIM), lambda i: (i, 0)),
            pl.BlockSpec((EDIM, H), lambda i: (0, 0)),
            pl.BlockSpec((1, H), lambda i: (0, 0)),
            pl.BlockSpec((H, H), lambda i: (0, 0)),
            pl.BlockSpec((1, H), lambda i: (0, 0)),
        ],
        out_specs=pl.BlockSpec((TE, H), lambda i: (i, 0)),
        out_shape=jax.ShapeDtypeStruct((E, H), jnp.float32),
    )(ea, w1, b1, w2, b2)


def _edgeN_body(g_ref, hp_ref, we_ref, b1_ref, w2_ref, b2_ref, o_ref):
    t = g_ref[...] + jnp.dot(
        hp_ref[...], we_ref[...], preferred_element_type=jnp.float32) + b1_ref[...]
    t = _silu(t)
    o_ref[...] = _silu(jnp.dot(t, w2_ref[...],
                               preferred_element_type=jnp.float32) + b2_ref[...])


def _edge_mlpN(g, hp, we, b1, w2, b2):
    return pl.pallas_call(
        _edgeN_body,
        grid=(E // TE,),
        in_specs=[
            pl.BlockSpec((TE, H), lambda i: (i, 0)),
            pl.BlockSpec((TE, H), lambda i: (i, 0)),
            pl.BlockSpec((H, H), lambda i: (0, 0)),
            pl.BlockSpec((1, H), lambda i: (0, 0)),
            pl.BlockSpec((H, H), lambda i: (0, 0)),
            pl.BlockSpec((1, H), lambda i: (0, 0)),
        ],
        out_specs=pl.BlockSpec((TE, H), lambda i: (i, 0)),
        out_shape=jax.ShapeDtypeStruct((E, H), jnp.float32),
    )(g, hp, we, b1, w2, b2)


def _node_mid_body(x_ref, s0_ref, s1_ref, c_ref, u1w_ref, u1b_ref,
                   u2w_ref, u2b_ref, ws_ref, wr_ref,
                   oxn_ref, oxs_ref, oxr_ref):
    c = jnp.maximum(c_ref[...][:, 0:1], 1.0)
    xm = x_ref[...] + (s0_ref[...] + s1_ref[...]) / c
    u = _silu(jnp.dot(xm, u1w_ref[...],
                      preferred_element_type=jnp.float32) + u1b_ref[...])
    xn = xm + jnp.dot(u, u2w_ref[...],
                      preferred_element_type=jnp.float32) + u2b_ref[...]
    oxn_ref[...] = xn
    oxs_ref[...] = jnp.dot(xn, ws_ref[...], preferred_element_type=jnp.float32)
    oxr_ref[...] = jnp.dot(xn, wr_ref[...], preferred_element_type=jnp.float32)


def _node_mid(x, s0, s1, cnt, u1w, u1b, u2w, u2b, ws, wr):
    return pl.pallas_call(
        _node_mid_body,
        grid=(N // TN,),
        in_specs=[
            pl.BlockSpec((TN, H), lambda i: (i, 0)),
            pl.BlockSpec((TN, H), lambda i: (i, 0)),
            pl.BlockSpec((TN, H), lambda i: (i, 0)),
            pl.BlockSpec((TN, H), lambda i: (i, 0)),
            pl.BlockSpec((H, 2 * H), lambda i: (0, 0)),
            pl.BlockSpec((1, 2 * H), lambda i: (0, 0)),
            pl.BlockSpec((2 * H, H), lambda i: (0, 0)),
            pl.BlockSpec((1, H), lambda i: (0, 0)),
            pl.BlockSpec((H, H), lambda i: (0, 0)),
            pl.BlockSpec((H, H), lambda i: (0, 0)),
        ],
        out_specs=[
            pl.BlockSpec((TN, H), lambda i: (i, 0)),
            pl.BlockSpec((TN, H), lambda i: (i, 0)),
            pl.BlockSpec((TN, H), lambda i: (i, 0)),
        ],
        out_shape=[
            jax.ShapeDtypeStruct((N, H), jnp.float32),
            jax.ShapeDtypeStruct((N, H), jnp.float32),
            jax.ShapeDtypeStruct((N, H), jnp.float32),
        ],
    )(x, s0, s1, cnt, u1w, u1b, u2w, u2b, ws, wr)


def _node_last_body(x_ref, s0_ref, s1_ref, c_ref, u1w_ref, u1b_ref,
                    u2w_ref, u2b_ref, w1_ref, b1_ref, w2_ref, b2_ref,
                    w3_ref, b3_ref, o_ref):
    c = jnp.maximum(c_ref[...][:, 0:1], 1.0)
    xm = x_ref[...] + (s0_ref[...] + s1_ref[...]) / c
    u = _silu(jnp.dot(xm, u1w_ref[...],
                      preferred_element_type=jnp.float32) + u1b_ref[...])
    xn = xm + jnp.dot(u, u2w_ref[...],
                      preferred_element_type=jnp.float32) + u2b_ref[...]
    t = _silu(jnp.dot(xn, w1_ref[...],
                      preferred_element_type=jnp.float32) + b1_ref[...])
    t = _silu(jnp.dot(t, w2_ref[...],
                      preferred_element_type=jnp.float32) + b2_ref[...])
    o_ref[...] = jnp.dot(t, w3_ref[...],
                         preferred_element_type=jnp.float32) + b3_ref[...]


def _node_last(x, s0, s1, cnt, u1w, u1b, u2w, u2b, w1, b1, w2, b2, w3, b3):
    return pl.pallas_call(
        _node_last_body,
        grid=(N // TN,),
        in_specs=[
            pl.BlockSpec((TN, H), lambda i: (i, 0)),
            pl.BlockSpec((TN, H), lambda i: (i, 0)),
            pl.BlockSpec((TN, H), lambda i: (i, 0)),
            pl.BlockSpec((TN, H), lambda i: (i, 0)),
            pl.BlockSpec((H, 2 * H), lambda i: (0, 0)),
            pl.BlockSpec((1, 2 * H), lambda i: (0, 0)),
            pl.BlockSpec((2 * H, H), lambda i: (0, 0)),
            pl.BlockSpec((1, H), lambda i: (0, 0)),
            pl.BlockSpec((H, H), lambda i: (0, 0)),
            pl.BlockSpec((1, H), lambda i: (0, 0)),
            pl.BlockSpec((H, H), lambda i: (0, 0)),
            pl.BlockSpec((1, H), lambda i: (0, 0)),
            pl.BlockSpec((H, OUT), lambda i: (0, 0)),
            pl.BlockSpec((1, OUT), lambda i: (0, 0)),
        ],
        out_specs=pl.BlockSpec((TN, OUT), lambda i: (i, 0)),
        out_shape=jax.ShapeDtypeStruct((N, OUT), jnp.float32),
    )(x, s0, s1, cnt, u1w, u1b, u2w, u2b, w1, b1, w2, b2, w3, b3)


def kernel(x, edge_attr, edges, params):
    send, recv = edges[0], edges[1]
    p1 = params["l1"]
    plist = [params["l2"], params["l3"], params["l4"]]
    po = params["out"]

    def r1(b):
        return b.reshape(1, -1)

    h = _edge_mlp1(edge_attr, p1["m1w"], r1(p1["m1b"]),
                   p1["m2w"], r1(p1["m2b"]))
    cnts = _sc_counts(recv)
    cnt = cnts[:N] + cnts[N:]
    sums = _sc_scatter(h, recv)
    s0, s1 = sums[:N], sums[N:]

    x_cur = x
    ulayer = p1
    for pn in plist:
        ws, wr, we = (pn["m1w"][:H], pn["m1w"][H:2 * H], pn["m1w"][2 * H:])
        x_cur, xs, xr = _node_mid(x_cur, s0, s1, cnt,
                                  ulayer["u1w"], r1(ulayer["u1b"]),
                                  ulayer["u2w"], r1(ulayer["u2b"]), ws, wr)
        g = _sc_gather(xs, xr, send, recv)
        h = _edge_mlpN(g, h, we, r1(pn["m1b"]),
                       pn["m2w"], r1(pn["m2b"]))
        sums = _sc_scatter(h, recv)
        s0, s1 = sums[:N], sums[N:]
        ulayer = pn

    return _node_last(x_cur, s0, s1, cnt,
                      ulayer["u1w"], r1(ulayer["u1b"]),
                      ulayer["u2w"], r1(ulayer["u2b"]),
                      po["w1"], r1(po["b1"]), po["w2"], r1(po["b2"]),
                      po["w3"], r1(po["b3"]))
